# baseline (speedup 1.0000x reference)
import functools

import jax
import jax.numpy as jnp
from jax import lax
from jax.experimental import pallas as pl
from jax.experimental.pallas import tpu as pltpu
from jax.experimental.pallas import tpu_sc as plsc

NC = 2
NS = 16
NW = NC * NS
L = 16

CHUNK = 256
SUB = 32
NSUB = CHUNK // SUB
NBUF = 4

_BLK = 32
_QMAX = 2047.0
_RND = 12582912.0


def _tree_max(vs):
    while len(vs) > 1:
        vs = [jnp.maximum(vs[i], vs[i + 1]) for i in range(0, len(vs) - 1, 2)] + (
            [vs[-1]] if len(vs) % 2 else []
        )
    return vs[0]


def _quant_group(rows_v, rb):
    riota = rb + lax.iota(jnp.int32, L)
    lane = lax.iota(jnp.int32, L)
    for blk in range(2):
        cidx = [blk * _BLK + ((lane + c) & (_BLK - 1)) for c in range(_BLK)]
        vals = [plsc.load_gather(rows_v, [riota, cidx[c]]) for c in range(_BLK)]
        am = _tree_max([jnp.abs(v) for v in vals])
        ms = jnp.maximum(am, 1e-30)
        inv = _QMAX / ms
        scale = ms * (1.0 / _QMAX)
        for c in range(_BLK):
            q = (vals[c] * inv + _RND) - _RND
            plsc.store_scatter(rows_v, [riota, cidx[c]], q * scale)


def _make_kernel(N, D):
    per_w = N // NW
    nchunks = per_w // CHUNK
    nquads = nchunks // NBUF
    mesh = plsc.VectorSubcoreMesh(core_axis_name="c", subcore_axis_name="s")

    @functools.partial(
        pl.kernel,
        mesh=mesh,
        compiler_params=pltpu.CompilerParams(
            needs_layout_passes=False, use_tc_tiling_on_sc=False
        ),
        out_type=jax.ShapeDtypeStruct((N, D), jnp.float32),
        scratch_types=[
            pltpu.VMEM((NBUF, NSUB, SUB), jnp.int32),
            pltpu.VMEM((NBUF, CHUNK, D), jnp.float32),
            pltpu.SemaphoreType.DMA((NBUF,)),
            pltpu.SemaphoreType.DMA((NBUF,)),
        ],
    )
    def k(idx_hbm, tab_hbm, out_hbm, idx_v, rows_v, gsem, osem):
        wid = lax.axis_index("s") * NC + lax.axis_index("c")
        sub0 = wid * (per_w // SUB)
        row0 = wid * per_w

        def stage_and_fire(g, b):
            pltpu.sync_copy(idx_hbm.at[pl.ds(sub0 + g * NSUB, NSUB)], idx_v.at[b])
            for j in range(NSUB):
                pltpu.make_async_copy(
                    tab_hbm.at[idx_v.at[b, j]],
                    rows_v.at[b, pl.ds(j * SUB, SUB)],
                    gsem.at[b],
                ).start()

        def wait_gather(b):
            for j in range(NSUB):
                pltpu.make_async_copy(
                    tab_hbm.at[idx_v.at[b, j]],
                    rows_v.at[b, pl.ds(j * SUB, SUB)],
                    gsem.at[b],
                ).wait()

        def out_copy(g, b):
            return pltpu.make_async_copy(
                rows_v.at[b], out_hbm.at[pl.ds(row0 + g * CHUNK, CHUNK)], osem.at[b]
            )

        def compute(b):
            @plsc.parallel_loop(0, CHUNK // L, 1, unroll=2)
            def _(i):
                _quant_group(rows_v.at[b], i * L)

        stage_and_fire(0, 0)
        stage_and_fire(1, 1)

        def quad_body(p, carry):
            for b in range(NBUF):
                g = NBUF * p + b
                nb = (b + 2) % NBUF

                @pl.when(g + 2 < nchunks)
                def _():
                    @pl.when(g >= 2)
                    def _():
                        out_copy(g - 2, nb).wait()

                    stage_and_fire(g + 2, nb)

                wait_gather(b)
                compute(b)
                out_copy(g, b).start()
            return carry

        lax.fori_loop(0, nquads, quad_body, 0)
        for b in range(NBUF):
            out_copy(nchunks - NBUF + b, b).wait()

    return k


def kernel(indices, weight):
    B, H = indices.shape
    V, D = weight.shape
    N = B * H
    idx2d = indices.reshape(N // SUB, SUB).astype(jnp.int32)
    out = _make_kernel(N, D)(idx2d, weight)
    return out.reshape(B, H, D)


# speedup vs baseline: 1.1027x; 1.1027x over previous
import functools

import jax
import jax.numpy as jnp
from jax import lax
from jax.experimental import pallas as pl
from jax.experimental.pallas import tpu as pltpu
from jax.experimental.pallas import tpu_sc as plsc

NC = 2
NS = 16
NW = NC * NS
L = 16

CHUNK = 128
NBUF = 4

_BLK = 32
_QMAX = 2047.0
_RND = 12582912.0


def _tree_max(vs):
    while len(vs) > 1:
        vs = [jnp.maximum(vs[i], vs[i + 1]) for i in range(0, len(vs) - 1, 2)] + (
            [vs[-1]] if len(vs) % 2 else []
        )
    return vs[0]


def _quant_group(rows_v, outs_v, idx_v, rb):
    riota = rb + lax.iota(jnp.int32, L)
    lane = lax.iota(jnp.int32, L)
    orow = lax.shift_right_logical(riota, 1)
    iv = idx_v[pl.ds(rb, L)]
    hin = lax.shift_left(iv & 1, 6)
    hout = lax.shift_left(lane & 1, 6)
    for blk in range(2):
        skew = [blk * _BLK + ((lane + c) & (_BLK - 1)) for c in range(_BLK)]
        icol = [hin + skew[c] for c in range(_BLK)]
        vals = [plsc.load_gather(rows_v, [riota, icol[c]]) for c in range(_BLK)]
        am = _tree_max([jnp.abs(v) for v in vals])
        ms = jnp.maximum(am, 1e-30)
        inv = _QMAX / ms
        scale = ms * (1.0 / _QMAX)
        for c in range(_BLK):
            q = (vals[c] * inv + _RND) - _RND
            plsc.store_scatter(outs_v, [orow, hout + skew[c]], q * scale)


def _make_kernel(N, D):
    per_w = N // NW
    nchunks = per_w // CHUNK
    nquads = nchunks // NBUF
    D2 = 2 * D
    mesh = plsc.VectorSubcoreMesh(core_axis_name="c", subcore_axis_name="s")

    @functools.partial(
        pl.kernel,
        mesh=mesh,
        compiler_params=pltpu.CompilerParams(
            needs_layout_passes=False, use_tc_tiling_on_sc=True
        ),
        out_type=jax.ShapeDtypeStruct((N // 2, D2), jnp.float32),
        scratch_types=[
            pltpu.VMEM((NBUF, CHUNK), jnp.int32),
            pltpu.VMEM((NBUF, CHUNK), jnp.int32),
            pltpu.VMEM((NBUF, CHUNK, D2), jnp.float32),
            pltpu.VMEM((NBUF, CHUNK // 2, D2), jnp.float32),
            pltpu.SemaphoreType.DMA((NBUF,)),
            pltpu.SemaphoreType.DMA((NBUF,)),
        ],
    )
    def k(idx_hbm, tab_hbm, out_hbm, idx_v, fidx_v, rows_v, outs_v, gsem, osem):
        wid = lax.axis_index("s") * NC + lax.axis_index("c")
        chunk0 = wid * nchunks
        orow0 = wid * (per_w // 2)

        def stage_and_fire(g, b):
            pltpu.sync_copy(idx_hbm.at[chunk0 + g], idx_v.at[b])
            for t in range(CHUNK // L):
                fidx_v[b, pl.ds(t * L, L)] = lax.shift_right_logical(
                    idx_v[b, pl.ds(t * L, L)], 1
                )
            pltpu.make_async_copy(
                tab_hbm.at[fidx_v.at[b]], rows_v.at[b], gsem.at[b]
            ).start()

        def wait_gather(b):
            pltpu.make_async_copy(
                tab_hbm.at[fidx_v.at[b]], rows_v.at[b], gsem.at[b]
            ).wait()

        def out_copy(g, b):
            return pltpu.make_async_copy(
                outs_v.at[b],
                out_hbm.at[pl.ds(orow0 + g * (CHUNK // 2), CHUNK // 2)],
                osem.at[b],
            )

        def compute(b):
            @plsc.parallel_loop(0, CHUNK // L, 1, unroll=2)
            def _(i):
                _quant_group(rows_v.at[b], outs_v.at[b], idx_v.at[b], i * L)

        stage_and_fire(0, 0)
        stage_and_fire(1, 1)

        def quad_body(p, carry):
            for b in range(NBUF):
                g = NBUF * p + b
                nb = (b + 2) % NBUF

                @pl.when(g + 2 < nchunks)
                def _():
                    @pl.when(g >= 2)
                    def _():
                        out_copy(g - 2, nb).wait()

                    stage_and_fire(g + 2, nb)

                wait_gather(b)
                compute(b)
                out_copy(g, b).start()
            return carry

        lax.fori_loop(0, nquads, quad_body, 0)
        for b in range(NBUF):
            out_copy(nchunks - NBUF + b, b).wait()

    return k


def kernel(indices, weight):
    B, H = indices.shape
    V, D = weight.shape
    N = B * H
    idx2d = indices.reshape(N // CHUNK, CHUNK).astype(jnp.int32)
    wfused = weight.reshape(V // 2, 2 * D)
    out = _make_kernel(N, D)(idx2d, wfused)
    return out.reshape(B, H, D)
